# Initial kernel scaffold; baseline (speedup 1.0000x reference)
import jax
import jax.numpy as jnp
from jax.experimental import pallas as pl


def kernel(boxes, scores, classes):
    raise NotImplementedError("write your pallas kernel here")


# speedup vs baseline: 4.2511x; 4.2511x over previous
import functools

import jax
import jax.numpy as jnp
from jax.experimental import pallas as pl
from jax.experimental.pallas import tpu as pltpu

_LINKAGE_TH = 0.2
_IOU_TH = 0.2
_N_ITER = 20
_ROW_CHUNK = 320


def _chunks(n, c):
    out, r = [], 0
    while r < n:
        s = min(c, n - r)
        out.append((r, s))
        r += s
    return out


def _iou_block(ax1, ay1, ax2, ay2, bx1, by1, bx2, by2):
    area_a = (ax2 - ax1) * (ay2 - ay1)
    area_b = (bx2 - bx1) * (by2 - by1)
    ltx = jnp.maximum(ax1, bx1)
    lty = jnp.maximum(ay1, by1)
    rbx = jnp.minimum(ax2, bx2)
    rby = jnp.minimum(ay2, by2)
    w = jnp.maximum(rbx - ltx, 0.0)
    h = jnp.maximum(rby - lty, 0.0)
    inter = w * h
    return inter / (area_a + area_b - inter + 1e-8)


def _iota(shape, axis):
    return jax.lax.broadcasted_iota(jnp.int32, shape, axis)


def _seqnms_kernel(x1, y1, x2, y2, x1t, y1t, x2t, y2t, cls, clst, sc, sct,
                   out_ref, mask_ref, *, T, N):
    X1, Y1, X2, Y2 = x1[...], y1[...], x2[...], y2[...]
    X1T, Y1T, X2T, Y2T = x1t[...], y1t[...], x2t[...], y2t[...]
    CLS, CLST = cls[...], clst[...]
    SC, SCT = sc[...], sct[...]
    chunks = _chunks(N, _ROW_CHUNK)

    for k in range(T - 1):
        rf, lf = (k + 1, k) if k % 2 == 0 else (k, k + 1)
        bx1 = X1[lf:lf + 1, :]
        by1 = Y1[lf:lf + 1, :]
        bx2 = X2[lf:lf + 1, :]
        by2 = Y2[lf:lf + 1, :]
        bcl = CLS[lf:lf + 1, :]
        for (r0, szz) in chunks:
            ax1 = X1T[r0:r0 + szz, rf:rf + 1]
            ay1 = Y1T[r0:r0 + szz, rf:rf + 1]
            ax2 = X2T[r0:r0 + szz, rf:rf + 1]
            ay2 = Y2T[r0:r0 + szz, rf:rf + 1]
            acl = CLST[r0:r0 + szz, rf:rf + 1]
            iou = _iou_block(ax1, ay1, ax2, ay2, bx1, by1, bx2, by2)
            m = (iou >= _LINKAGE_TH) & (acl == bcl)
            mask_ref[k, pl.ds(r0, szz), :] = m.astype(jnp.int8)

    def iter_body(_, carry):
        cand, cand2, out = carry

        cums = [cand[0:1, :]]
        for t in range(1, T):
            k = t - 1
            if k % 2 == 0:
                prev = cums[k]
                pbs = []
                for (r0, szz) in chunks:
                    mc = mask_ref[k, pl.ds(r0, szz), :].astype(jnp.float32)
                    pbs.append(jnp.max(mc * prev, axis=1, keepdims=True))
                pb = jnp.concatenate(pbs, axis=0)
                cums.append(cand2[:, t:t + 1] + pb)
            else:
                prev = cums[k]
                acc = jnp.zeros((1, N), jnp.float32)
                for (r0, szz) in chunks:
                    mc = mask_ref[k, pl.ds(r0, szz), :].astype(jnp.float32)
                    acc = jnp.maximum(
                        acc, jnp.max(mc * prev[r0:r0 + szz, :], axis=0,
                                     keepdims=True))
                cums.append(cand[t:t + 1, :] + acc)

        maxes, idxs = [], []
        for t in range(T):
            v = cums[t]
            ax = 1 if t % 2 == 0 else 0
            m = jnp.max(v, axis=(0, 1), keepdims=True)
            io = _iota(v.shape, ax)
            ix = jnp.min(jnp.where(v == m, io, N), axis=(0, 1), keepdims=True)
            maxes.append(m)
            idxs.append(ix)
        best = maxes[0]
        bi = jnp.zeros((1, 1), jnp.int32)
        for t in range(1, T):
            c = maxes[t] > best
            bi = jnp.where(c, t, bi)
            best = jnp.where(c, maxes[t], best)
        istar = jnp.zeros((1, 1), jnp.int32)
        for t in range(T):
            istar = jnp.where(bi == t, idxs[t], istar)
        iter_ok = best > 0.0

        neg1 = jnp.full((1, 1), -1, jnp.int32)
        active = jnp.zeros((1, 1), jnp.bool_)
        cur = jnp.zeros((1, 1), jnp.int32)
        sels = [None] * T
        for t in range(T - 1, -1, -1):
            here = bi == t
            active = jnp.logical_or(active, here)
            cur = jnp.where(here, istar, cur)
            sels[t] = jnp.where(active, cur, neg1)
            if t > 0:
                k = t - 1
                curc = jnp.maximum(cur, 0)
                ohr = _iota((1, N), 1) == curc
                ohf = ohr.astype(jnp.float32)
                ohi = ohr.astype(jnp.int32)
                sx1 = jnp.sum(X1[t:t + 1, :] * ohf, axis=1, keepdims=True)
                sy1 = jnp.sum(Y1[t:t + 1, :] * ohf, axis=1, keepdims=True)
                sx2 = jnp.sum(X2[t:t + 1, :] * ohf, axis=1, keepdims=True)
                sy2 = jnp.sum(Y2[t:t + 1, :] * ohf, axis=1, keepdims=True)
                scl = jnp.sum(CLS[t:t + 1, :] * ohi, axis=1, keepdims=True)
                if k % 2 == 0:
                    fx1, fy1 = X1[k:k + 1, :], Y1[k:k + 1, :]
                    fx2, fy2 = X2[k:k + 1, :], Y2[k:k + 1, :]
                    fcl = CLS[k:k + 1, :]
                    ax = 1
                else:
                    fx1, fy1 = X1T[:, k:k + 1], Y1T[:, k:k + 1]
                    fx2, fy2 = X2T[:, k:k + 1], Y2T[:, k:k + 1]
                    fcl = CLST[:, k:k + 1]
                    ax = 0
                prev = cums[k]
                iouv = _iou_block(fx1, fy1, fx2, fy2, sx1, sy1, sx2, sy2)
                linkv = (iouv >= _LINKAGE_TH) & (fcl == scl)
                pbv = jnp.max(jnp.where(linkv, prev, 0.0), axis=(0, 1),
                              keepdims=True)
                masked = jnp.where(linkv, prev, -1.0)
                mm = jnp.max(masked, axis=(0, 1), keepdims=True)
                io = _iota(masked.shape, ax)
                pidx = jnp.min(jnp.where(masked == mm, io, N), axis=(0, 1),
                               keepdims=True)
                par = jnp.where(pbv > 0.0, pidx, neg1)
                active = jnp.logical_and(active, par >= 0)
                cur = jnp.where(active, par, cur)

        selc = jnp.zeros((T, 1), jnp.int32)
        selr = jnp.zeros((1, T), jnp.int32)
        for t in range(T):
            selc = jnp.where(_iota((T, 1), 0) == t, sels[t], selc)
            selr = jnp.where(_iota((1, T), 1) == t, sels[t], selr)
        validc = (selc >= 0) & iter_ok
        validr = (selr >= 0) & iter_ok
        idc = jnp.maximum(selc, 0)
        idr = jnp.maximum(selr, 0)

        oh = _iota((T, N), 1) == idc
        ohf = oh.astype(jnp.float32)
        ohi = oh.astype(jnp.int32)
        vf = validc.astype(jnp.float32)
        g = jnp.sum(cand * ohf, axis=1, keepdims=True) * vf
        cnt = jnp.maximum(jnp.sum(vf, axis=(0, 1), keepdims=True), 1.0)
        avg = jnp.sum(g, axis=(0, 1), keepdims=True) / cnt
        out = jnp.where(validc & oh, avg, out)

        scl = jnp.sum(CLS * ohi, axis=1, keepdims=True)
        sx1 = jnp.sum(X1 * ohf, axis=1, keepdims=True)
        sy1 = jnp.sum(Y1 * ohf, axis=1, keepdims=True)
        sx2 = jnp.sum(X2 * ohf, axis=1, keepdims=True)
        sy2 = jnp.sum(Y2 * ohf, axis=1, keepdims=True)
        iour = _iou_block(sx1, sy1, sx2, sy2, X1, Y1, X2, Y2)
        supp = validc & (iour >= _IOU_TH) & (CLS == scl)
        cand = jnp.where(supp, 0.0, cand)

        ohT = _iota((N, T), 0) == idr
        ohTf = ohT.astype(jnp.float32)
        ohTi = ohT.astype(jnp.int32)
        sclT = jnp.sum(CLST * ohTi, axis=0, keepdims=True)
        sxT1 = jnp.sum(X1T * ohTf, axis=0, keepdims=True)
        syT1 = jnp.sum(Y1T * ohTf, axis=0, keepdims=True)
        sxT2 = jnp.sum(X2T * ohTf, axis=0, keepdims=True)
        syT2 = jnp.sum(Y2T * ohTf, axis=0, keepdims=True)
        iouT = _iou_block(sxT1, syT1, sxT2, syT2, X1T, Y1T, X2T, Y2T)
        suppT = validr & (iouT >= _IOU_TH) & (CLST == sclT)
        cand2 = jnp.where(suppT, 0.0, cand2)

        return cand, cand2, out

    init = (SC, SCT, jnp.zeros((T, N), jnp.float32))
    _, _, out = jax.lax.fori_loop(0, _N_ITER, iter_body, init)
    out_ref[...] = out


def _build(T, N):
    return pl.pallas_call(
        functools.partial(_seqnms_kernel, T=T, N=N),
        out_shape=jax.ShapeDtypeStruct((T, N), jnp.float32),
        scratch_shapes=[pltpu.VMEM((T - 1, N, N), jnp.int8)],
    )


@jax.jit
def kernel(boxes, scores, classes):
    T, N = scores.shape
    x1 = boxes[..., 0]
    y1 = boxes[..., 1]
    x2 = boxes[..., 2]
    y2 = boxes[..., 3]
    return _build(T, N)(
        x1, y1, x2, y2,
        x1.T, y1.T, x2.T, y2.T,
        classes, classes.T,
        scores, scores.T,
    )


# speedup vs baseline: 4.7063x; 1.1071x over previous
import functools

import jax
import jax.numpy as jnp
from jax.experimental import pallas as pl
from jax.experimental.pallas import tpu as pltpu

_LINKAGE_TH = 0.2
_IOU_TH = 0.2
_N_ITER = 20
_ROW_CHUNK = 320


def _chunks(n, c):
    out, r = [], 0
    while r < n:
        s = min(c, n - r)
        out.append((r, s))
        r += s
    return out


def _iou_block(ax1, ay1, ax2, ay2, bx1, by1, bx2, by2):
    area_a = (ax2 - ax1) * (ay2 - ay1)
    area_b = (bx2 - bx1) * (by2 - by1)
    ltx = jnp.maximum(ax1, bx1)
    lty = jnp.maximum(ay1, by1)
    rbx = jnp.minimum(ax2, bx2)
    rby = jnp.minimum(ay2, by2)
    w = jnp.maximum(rbx - ltx, 0.0)
    h = jnp.maximum(rby - lty, 0.0)
    inter = w * h
    return inter / (area_a + area_b - inter + 1e-8)


def _iota(shape, axis):
    return jax.lax.broadcasted_iota(jnp.int32, shape, axis)


def _seqnms_kernel(x1, y1, x2, y2, x1t, y1t, x2t, y2t, cls, clst, sc, sct,
                   out_ref, mask_ref, *, T, N):
    X1, Y1, X2, Y2 = x1[...], y1[...], x2[...], y2[...]
    X1T, Y1T, X2T, Y2T = x1t[...], y1t[...], x2t[...], y2t[...]
    CLS, CLST = cls[...], clst[...]
    SC, SCT = sc[...], sct[...]
    chunks = _chunks(N, _ROW_CHUNK)

    for k in range(T - 1):
        rf, lf = (k + 1, k) if k % 2 == 0 else (k, k + 1)
        bx1 = X1[lf:lf + 1, :]
        by1 = Y1[lf:lf + 1, :]
        bx2 = X2[lf:lf + 1, :]
        by2 = Y2[lf:lf + 1, :]
        bcl = CLS[lf:lf + 1, :]
        for (r0, szz) in chunks:
            ax1 = X1T[r0:r0 + szz, rf:rf + 1]
            ay1 = Y1T[r0:r0 + szz, rf:rf + 1]
            ax2 = X2T[r0:r0 + szz, rf:rf + 1]
            ay2 = Y2T[r0:r0 + szz, rf:rf + 1]
            acl = CLST[r0:r0 + szz, rf:rf + 1]
            iou = _iou_block(ax1, ay1, ax2, ay2, bx1, by1, bx2, by2)
            m = (iou >= _LINKAGE_TH) & (acl == bcl)
            mask_ref[k, pl.ds(r0, szz), :] = m.astype(jnp.int8)

    def iter_body(_, carry):
        cand, cand2, out = carry

        cums = [cand[0:1, :]]
        for t in range(1, T):
            k = t - 1
            if k % 2 == 0:
                prev = cums[k]
                pbs = []
                for (r0, szz) in chunks:
                    mc = mask_ref[k, pl.ds(r0, szz), :].astype(jnp.float32)
                    pbs.append(jnp.max(mc * prev, axis=1, keepdims=True))
                pb = jnp.concatenate(pbs, axis=0)
                cums.append(cand2[:, t:t + 1] + pb)
            else:
                prev = cums[k]
                acc = jnp.zeros((1, N), jnp.float32)
                for (r0, szz) in chunks:
                    mc = mask_ref[k, pl.ds(r0, szz), :].astype(jnp.float32)
                    acc = jnp.maximum(
                        acc, jnp.max(mc * prev[r0:r0 + szz, :], axis=0,
                                     keepdims=True))
                cums.append(cand[t:t + 1, :] + acc)

        maxes, idxs = [], []
        for t in range(T):
            v = cums[t]
            ax = 1 if t % 2 == 0 else 0
            m = jnp.max(v, axis=(0, 1), keepdims=True)
            io = _iota(v.shape, ax)
            ix = jnp.min(jnp.where(v == m, io, N), axis=(0, 1), keepdims=True)
            maxes.append(m)
            idxs.append(ix)
        best = maxes[0]
        bi = jnp.zeros((1, 1), jnp.int32)
        for t in range(1, T):
            c = maxes[t] > best
            bi = jnp.where(c, t, bi)
            best = jnp.where(c, maxes[t], best)
        istar = jnp.zeros((1, 1), jnp.int32)
        for t in range(T):
            istar = jnp.where(bi == t, idxs[t], istar)
        iter_ok = best > 0.0

        cums_row = [cums[k] if k % 2 == 0 else jnp.transpose(cums[k])
                    for k in range(T - 1)]

        neg1 = jnp.full((1, 1), -1, jnp.int32)
        active = jnp.zeros((1, 1), jnp.bool_)
        cur = jnp.zeros((1, 1), jnp.int32)
        sels = [None] * T
        for t in range(T - 1, -1, -1):
            here = bi == t
            active = jnp.logical_or(active, here)
            cur = jnp.where(here, istar, cur)
            sels[t] = jnp.where(active, cur, neg1)
            if t > 0:
                k = t - 1
                curc = jnp.maximum(cur, 0)
                ohr = _iota((1, N), 1) == curc
                ohf = ohr.astype(jnp.float32)
                ohi = ohr.astype(jnp.int32)
                sx1 = jnp.sum(X1[t:t + 1, :] * ohf, axis=1, keepdims=True)
                sy1 = jnp.sum(Y1[t:t + 1, :] * ohf, axis=1, keepdims=True)
                sx2 = jnp.sum(X2[t:t + 1, :] * ohf, axis=1, keepdims=True)
                sy2 = jnp.sum(Y2[t:t + 1, :] * ohf, axis=1, keepdims=True)
                scl = jnp.sum(CLS[t:t + 1, :] * ohi, axis=1, keepdims=True)
                fx1, fy1 = X1[k:k + 1, :], Y1[k:k + 1, :]
                fx2, fy2 = X2[k:k + 1, :], Y2[k:k + 1, :]
                fcl = CLS[k:k + 1, :]
                prev = cums_row[k]
                iouv = _iou_block(fx1, fy1, fx2, fy2, sx1, sy1, sx2, sy2)
                linkv = (iouv >= _LINKAGE_TH) & (fcl == scl)
                masked = jnp.where(linkv, prev, -1.0)
                mm = jnp.max(masked, axis=(0, 1), keepdims=True)
                io = _iota(masked.shape, 1)
                pidx = jnp.min(jnp.where(masked == mm, io, N), axis=(0, 1),
                               keepdims=True)
                par = jnp.where(mm > 0.0, pidx, neg1)
                active = jnp.logical_and(active, par >= 0)
                cur = jnp.where(active, par, cur)

        selc = jnp.zeros((T, 1), jnp.int32)
        selr = jnp.zeros((1, T), jnp.int32)
        for t in range(T):
            selc = jnp.where(_iota((T, 1), 0) == t, sels[t], selc)
            selr = jnp.where(_iota((1, T), 1) == t, sels[t], selr)
        validc = (selc >= 0) & iter_ok
        validr = (selr >= 0) & iter_ok
        idc = jnp.maximum(selc, 0)
        idr = jnp.maximum(selr, 0)

        oh = _iota((T, N), 1) == idc
        ohf = oh.astype(jnp.float32)
        ohi = oh.astype(jnp.int32)
        vf = validc.astype(jnp.float32)
        g = jnp.sum(cand * ohf, axis=1, keepdims=True) * vf
        cnt = jnp.maximum(jnp.sum(vf, axis=(0, 1), keepdims=True), 1.0)
        avg = jnp.sum(g, axis=(0, 1), keepdims=True) / cnt
        out = jnp.where(validc & oh, avg, out)

        scl = jnp.sum(CLS * ohi, axis=1, keepdims=True)
        sx1 = jnp.sum(X1 * ohf, axis=1, keepdims=True)
        sy1 = jnp.sum(Y1 * ohf, axis=1, keepdims=True)
        sx2 = jnp.sum(X2 * ohf, axis=1, keepdims=True)
        sy2 = jnp.sum(Y2 * ohf, axis=1, keepdims=True)
        iour = _iou_block(sx1, sy1, sx2, sy2, X1, Y1, X2, Y2)
        supp = validc & (iour >= _IOU_TH) & (CLS == scl)
        cand = jnp.where(supp, 0.0, cand)

        ohT = _iota((N, T), 0) == idr
        ohTf = ohT.astype(jnp.float32)
        ohTi = ohT.astype(jnp.int32)
        sclT = jnp.sum(CLST * ohTi, axis=0, keepdims=True)
        sxT1 = jnp.sum(X1T * ohTf, axis=0, keepdims=True)
        syT1 = jnp.sum(Y1T * ohTf, axis=0, keepdims=True)
        sxT2 = jnp.sum(X2T * ohTf, axis=0, keepdims=True)
        syT2 = jnp.sum(Y2T * ohTf, axis=0, keepdims=True)
        iouT = _iou_block(sxT1, syT1, sxT2, syT2, X1T, Y1T, X2T, Y2T)
        suppT = validr & (iouT >= _IOU_TH) & (CLST == sclT)
        cand2 = jnp.where(suppT, 0.0, cand2)

        return cand, cand2, out

    init = (SC, SCT, jnp.zeros((T, N), jnp.float32))
    _, _, out = jax.lax.fori_loop(0, _N_ITER, iter_body, init)
    out_ref[...] = out


def _build(T, N):
    return pl.pallas_call(
        functools.partial(_seqnms_kernel, T=T, N=N),
        out_shape=jax.ShapeDtypeStruct((T, N), jnp.float32),
        scratch_shapes=[pltpu.VMEM((T - 1, N, N), jnp.int8)],
    )


@jax.jit
def kernel(boxes, scores, classes):
    T, N = scores.shape
    x1 = boxes[..., 0]
    y1 = boxes[..., 1]
    x2 = boxes[..., 2]
    y2 = boxes[..., 3]
    return _build(T, N)(
        x1, y1, x2, y2,
        x1.T, y1.T, x2.T, y2.T,
        classes, classes.T,
        scores, scores.T,
    )


# speedup vs baseline: 5.2220x; 1.1096x over previous
import functools

import jax
import jax.numpy as jnp
from jax.experimental import pallas as pl
from jax.experimental.pallas import tpu as pltpu

_LINKAGE_TH = 0.2
_IOU_TH = 0.2
_N_ITER = 20
_ROW_CHUNK = 320


def _chunks(n, c):
    out, r = [], 0
    while r < n:
        s = min(c, n - r)
        out.append((r, s))
        r += s
    return out


def _iou_block(ax1, ay1, ax2, ay2, bx1, by1, bx2, by2):
    area_a = (ax2 - ax1) * (ay2 - ay1)
    area_b = (bx2 - bx1) * (by2 - by1)
    ltx = jnp.maximum(ax1, bx1)
    lty = jnp.maximum(ay1, by1)
    rbx = jnp.minimum(ax2, bx2)
    rby = jnp.minimum(ay2, by2)
    w = jnp.maximum(rbx - ltx, 0.0)
    h = jnp.maximum(rby - lty, 0.0)
    inter = w * h
    return inter / (area_a + area_b - inter + 1e-8)


def _iota(shape, axis):
    return jax.lax.broadcasted_iota(jnp.int32, shape, axis)


def _seqnms_kernel(x1, y1, x2, y2, x1t, y1t, x2t, y2t, cls, clst, sc,
                   out_ref, mask_ref, *, T, N):
    X1, Y1, X2, Y2 = x1[...], y1[...], x2[...], y2[...]
    X1T, Y1T, X2T, Y2T = x1t[...], y1t[...], x2t[...], y2t[...]
    CLS, CLST = cls[...], clst[...]
    SC = sc[...]
    chunks = _chunks(N, _ROW_CHUNK)

    for k in range(T - 1):
        rf, lf = k, k + 1
        bx1 = X1[lf:lf + 1, :]
        by1 = Y1[lf:lf + 1, :]
        bx2 = X2[lf:lf + 1, :]
        by2 = Y2[lf:lf + 1, :]
        bcl = CLS[lf:lf + 1, :]
        for (r0, szz) in chunks:
            ax1 = X1T[r0:r0 + szz, rf:rf + 1]
            ay1 = Y1T[r0:r0 + szz, rf:rf + 1]
            ax2 = X2T[r0:r0 + szz, rf:rf + 1]
            ay2 = Y2T[r0:r0 + szz, rf:rf + 1]
            acl = CLST[r0:r0 + szz, rf:rf + 1]
            iou = _iou_block(ax1, ay1, ax2, ay2, bx1, by1, bx2, by2)
            m = (iou >= _LINKAGE_TH) & (acl == bcl)
            mask_ref[k, pl.ds(r0, szz), :] = m.astype(jnp.int8)

    def iter_body(_, carry):
        cand, out = carry

        cums = [cand[0:1, :]]
        for t in range(1, T):
            k = t - 1
            prev = jnp.transpose(cums[k])
            acc = jnp.zeros((1, N), jnp.float32)
            for (r0, szz) in chunks:
                mc = mask_ref[k, pl.ds(r0, szz), :].astype(jnp.float32)
                acc = jnp.maximum(
                    acc, jnp.max(mc * prev[r0:r0 + szz, :], axis=0,
                                 keepdims=True))
            cums.append(cand[t:t + 1, :] + acc)

        maxes, idxs = [], []
        for t in range(T):
            v = cums[t]
            m = jnp.max(v, axis=(0, 1), keepdims=True)
            io = _iota(v.shape, 1)
            ix = jnp.min(jnp.where(v == m, io, N), axis=(0, 1), keepdims=True)
            maxes.append(m)
            idxs.append(ix)
        best = maxes[0]
        bi = jnp.zeros((1, 1), jnp.int32)
        for t in range(1, T):
            c = maxes[t] > best
            bi = jnp.where(c, t, bi)
            best = jnp.where(c, maxes[t], best)
        istar = jnp.zeros((1, 1), jnp.int32)
        for t in range(T):
            istar = jnp.where(bi == t, idxs[t], istar)
        iter_ok = best > 0.0

        neg1 = jnp.full((1, 1), -1, jnp.int32)
        active = jnp.zeros((1, 1), jnp.bool_)
        cur = jnp.zeros((1, 1), jnp.int32)
        sels = [None] * T
        for t in range(T - 1, -1, -1):
            here = bi == t
            active = jnp.logical_or(active, here)
            cur = jnp.where(here, istar, cur)
            sels[t] = jnp.where(active, cur, neg1)
            if t > 0:
                k = t - 1
                curc = jnp.maximum(cur, 0)
                ohr = _iota((1, N), 1) == curc
                ohf = ohr.astype(jnp.float32)
                ohi = ohr.astype(jnp.int32)
                sx1 = jnp.sum(X1[t:t + 1, :] * ohf, axis=1, keepdims=True)
                sy1 = jnp.sum(Y1[t:t + 1, :] * ohf, axis=1, keepdims=True)
                sx2 = jnp.sum(X2[t:t + 1, :] * ohf, axis=1, keepdims=True)
                sy2 = jnp.sum(Y2[t:t + 1, :] * ohf, axis=1, keepdims=True)
                scl = jnp.sum(CLS[t:t + 1, :] * ohi, axis=1, keepdims=True)
                fx1, fy1 = X1[k:k + 1, :], Y1[k:k + 1, :]
                fx2, fy2 = X2[k:k + 1, :], Y2[k:k + 1, :]
                fcl = CLS[k:k + 1, :]
                prev = cums[k]
                iouv = _iou_block(fx1, fy1, fx2, fy2, sx1, sy1, sx2, sy2)
                linkv = (iouv >= _LINKAGE_TH) & (fcl == scl)
                masked = jnp.where(linkv, prev, -1.0)
                mm = jnp.max(masked, axis=(0, 1), keepdims=True)
                io = _iota(masked.shape, 1)
                pidx = jnp.min(jnp.where(masked == mm, io, N), axis=(0, 1),
                               keepdims=True)
                par = jnp.where(mm > 0.0, pidx, neg1)
                active = jnp.logical_and(active, par >= 0)
                cur = jnp.where(active, par, cur)

        selc = jnp.zeros((T, 1), jnp.int32)
        for t in range(T):
            selc = jnp.where(_iota((T, 1), 0) == t, sels[t], selc)
        validc = (selc >= 0) & iter_ok
        idc = jnp.maximum(selc, 0)

        oh = _iota((T, N), 1) == idc
        ohf = oh.astype(jnp.float32)
        ohi = oh.astype(jnp.int32)
        vf = validc.astype(jnp.float32)
        g = jnp.sum(cand * ohf, axis=1, keepdims=True) * vf
        cnt = jnp.maximum(jnp.sum(vf, axis=(0, 1), keepdims=True), 1.0)
        avg = jnp.sum(g, axis=(0, 1), keepdims=True) / cnt
        out = jnp.where(validc & oh, avg, out)

        scl = jnp.sum(CLS * ohi, axis=1, keepdims=True)
        sx1 = jnp.sum(X1 * ohf, axis=1, keepdims=True)
        sy1 = jnp.sum(Y1 * ohf, axis=1, keepdims=True)
        sx2 = jnp.sum(X2 * ohf, axis=1, keepdims=True)
        sy2 = jnp.sum(Y2 * ohf, axis=1, keepdims=True)
        iour = _iou_block(sx1, sy1, sx2, sy2, X1, Y1, X2, Y2)
        supp = validc & (iour >= _IOU_TH) & (CLS == scl)
        cand = jnp.where(supp, 0.0, cand)

        return cand, out

    init = (SC, jnp.zeros((T, N), jnp.float32))
    _, out = jax.lax.fori_loop(0, _N_ITER, iter_body, init)
    out_ref[...] = out


def _build(T, N):
    return pl.pallas_call(
        functools.partial(_seqnms_kernel, T=T, N=N),
        out_shape=jax.ShapeDtypeStruct((T, N), jnp.float32),
        scratch_shapes=[pltpu.VMEM((T - 1, N, N), jnp.int8)],
    )


@jax.jit
def kernel(boxes, scores, classes):
    T, N = scores.shape
    x1 = boxes[..., 0]
    y1 = boxes[..., 1]
    x2 = boxes[..., 2]
    y2 = boxes[..., 3]
    return _build(T, N)(
        x1, y1, x2, y2,
        x1.T, y1.T, x2.T, y2.T,
        classes, classes.T,
        scores,
    )


# speedup vs baseline: 5.6958x; 1.0907x over previous
import functools

import jax
import jax.numpy as jnp
from jax.experimental import pallas as pl
from jax.experimental.pallas import tpu as pltpu

_LINKAGE_TH = 0.2
_IOU_TH = 0.2
_N_ITER = 20
_ROW_CHUNK = 1024


def _chunks(n, c):
    out, r = [], 0
    while r < n:
        s = min(c, n - r)
        out.append((r, s))
        r += s
    return out


def _iou_block(ax1, ay1, ax2, ay2, bx1, by1, bx2, by2):
    area_a = (ax2 - ax1) * (ay2 - ay1)
    area_b = (bx2 - bx1) * (by2 - by1)
    ltx = jnp.maximum(ax1, bx1)
    lty = jnp.maximum(ay1, by1)
    rbx = jnp.minimum(ax2, bx2)
    rby = jnp.minimum(ay2, by2)
    w = jnp.maximum(rbx - ltx, 0.0)
    h = jnp.maximum(rby - lty, 0.0)
    inter = w * h
    return inter / (area_a + area_b - inter + 1e-8)


def _iota(shape, axis):
    return jax.lax.broadcasted_iota(jnp.int32, shape, axis)


def _seqnms_kernel(x1, y1, x2, y2, x1t, y1t, x2t, y2t, cls, clst, sc,
                   out_ref, mask_ref, *, T, N):
    X1, Y1, X2, Y2 = x1[...], y1[...], x2[...], y2[...]
    X1T, Y1T, X2T, Y2T = x1t[...], y1t[...], x2t[...], y2t[...]
    CLS, CLST = cls[...], clst[...]
    SC = sc[...]
    chunks = _chunks(N, _ROW_CHUNK)

    for k in range(T - 1):
        rf, lf = k, k + 1
        bx1 = X1[lf:lf + 1, :]
        by1 = Y1[lf:lf + 1, :]
        bx2 = X2[lf:lf + 1, :]
        by2 = Y2[lf:lf + 1, :]
        bcl = CLS[lf:lf + 1, :]
        for (r0, szz) in chunks:
            ax1 = X1T[r0:r0 + szz, rf:rf + 1]
            ay1 = Y1T[r0:r0 + szz, rf:rf + 1]
            ax2 = X2T[r0:r0 + szz, rf:rf + 1]
            ay2 = Y2T[r0:r0 + szz, rf:rf + 1]
            acl = CLST[r0:r0 + szz, rf:rf + 1]
            iou = _iou_block(ax1, ay1, ax2, ay2, bx1, by1, bx2, by2)
            m = (iou >= _LINKAGE_TH) & (acl == bcl)
            mask_ref[k, pl.ds(r0, szz), :] = m.astype(jnp.int8)

    def iter_body(_, carry):
        cand, out = carry

        cums = [cand[0:1, :]]
        for t in range(1, T):
            k = t - 1
            prev = jnp.transpose(cums[k])
            acc = jnp.zeros((1, N), jnp.float32)
            for (r0, szz) in chunks:
                mc = mask_ref[k, pl.ds(r0, szz), :].astype(jnp.float32)
                acc = jnp.maximum(
                    acc, jnp.max(mc * prev[r0:r0 + szz, :], axis=0,
                                 keepdims=True))
            cums.append(cand[t:t + 1, :] + acc)

        maxes, idxs = [], []
        for t in range(T):
            v = cums[t]
            m = jnp.max(v, axis=(0, 1), keepdims=True)
            io = _iota(v.shape, 1)
            ix = jnp.min(jnp.where(v == m, io, N), axis=(0, 1), keepdims=True)
            maxes.append(m)
            idxs.append(ix)
        best = maxes[0]
        bi = jnp.zeros((1, 1), jnp.int32)
        for t in range(1, T):
            c = maxes[t] > best
            bi = jnp.where(c, t, bi)
            best = jnp.where(c, maxes[t], best)
        istar = jnp.zeros((1, 1), jnp.int32)
        for t in range(T):
            istar = jnp.where(bi == t, idxs[t], istar)
        iter_ok = best > 0.0

        neg1 = jnp.full((1, 1), -1, jnp.int32)
        active = jnp.zeros((1, 1), jnp.bool_)
        cur = jnp.zeros((1, 1), jnp.int32)
        sels = [None] * T
        for t in range(T - 1, -1, -1):
            here = bi == t
            active = jnp.logical_or(active, here)
            cur = jnp.where(here, istar, cur)
            sels[t] = jnp.where(active, cur, neg1)
            if t > 0:
                k = t - 1
                curc = jnp.maximum(cur, 0)
                ohr = _iota((1, N), 1) == curc
                ohf = ohr.astype(jnp.float32)
                ohi = ohr.astype(jnp.int32)
                sx1 = jnp.sum(X1[t:t + 1, :] * ohf, axis=1, keepdims=True)
                sy1 = jnp.sum(Y1[t:t + 1, :] * ohf, axis=1, keepdims=True)
                sx2 = jnp.sum(X2[t:t + 1, :] * ohf, axis=1, keepdims=True)
                sy2 = jnp.sum(Y2[t:t + 1, :] * ohf, axis=1, keepdims=True)
                scl = jnp.sum(CLS[t:t + 1, :] * ohi, axis=1, keepdims=True)
                fx1, fy1 = X1[k:k + 1, :], Y1[k:k + 1, :]
                fx2, fy2 = X2[k:k + 1, :], Y2[k:k + 1, :]
                fcl = CLS[k:k + 1, :]
                prev = cums[k]
                iouv = _iou_block(fx1, fy1, fx2, fy2, sx1, sy1, sx2, sy2)
                linkv = (iouv >= _LINKAGE_TH) & (fcl == scl)
                masked = jnp.where(linkv, prev, -1.0)
                mm = jnp.max(masked, axis=(0, 1), keepdims=True)
                io = _iota(masked.shape, 1)
                pidx = jnp.min(jnp.where(masked == mm, io, N), axis=(0, 1),
                               keepdims=True)
                par = jnp.where(mm > 0.0, pidx, neg1)
                active = jnp.logical_and(active, par >= 0)
                cur = jnp.where(active, par, cur)

        selc = jnp.zeros((T, 1), jnp.int32)
        for t in range(T):
            selc = jnp.where(_iota((T, 1), 0) == t, sels[t], selc)
        validc = (selc >= 0) & iter_ok
        idc = jnp.maximum(selc, 0)

        oh = _iota((T, N), 1) == idc
        ohf = oh.astype(jnp.float32)
        ohi = oh.astype(jnp.int32)
        vf = validc.astype(jnp.float32)
        g = jnp.sum(cand * ohf, axis=1, keepdims=True) * vf
        cnt = jnp.maximum(jnp.sum(vf, axis=(0, 1), keepdims=True), 1.0)
        avg = jnp.sum(g, axis=(0, 1), keepdims=True) / cnt
        out = jnp.where(validc & oh, avg, out)

        scl = jnp.sum(CLS * ohi, axis=1, keepdims=True)
        sx1 = jnp.sum(X1 * ohf, axis=1, keepdims=True)
        sy1 = jnp.sum(Y1 * ohf, axis=1, keepdims=True)
        sx2 = jnp.sum(X2 * ohf, axis=1, keepdims=True)
        sy2 = jnp.sum(Y2 * ohf, axis=1, keepdims=True)
        iour = _iou_block(sx1, sy1, sx2, sy2, X1, Y1, X2, Y2)
        supp = validc & (iour >= _IOU_TH) & (CLS == scl)
        cand = jnp.where(supp, 0.0, cand)

        return cand, out

    init = (SC, jnp.zeros((T, N), jnp.float32))
    _, out = jax.lax.fori_loop(0, _N_ITER, iter_body, init)
    out_ref[...] = out


def _build(T, N):
    return pl.pallas_call(
        functools.partial(_seqnms_kernel, T=T, N=N),
        out_shape=jax.ShapeDtypeStruct((T, N), jnp.float32),
        scratch_shapes=[pltpu.VMEM((T - 1, N, N), jnp.int8)],
    )


@jax.jit
def kernel(boxes, scores, classes):
    T, N = scores.shape
    x1 = boxes[..., 0]
    y1 = boxes[..., 1]
    x2 = boxes[..., 2]
    y2 = boxes[..., 3]
    return _build(T, N)(
        x1, y1, x2, y2,
        x1.T, y1.T, x2.T, y2.T,
        classes, classes.T,
        scores,
    )
